# baseline (speedup 1.0000x reference)
import functools

import jax
import jax.numpy as jnp
from jax import lax
from jax.experimental import pallas as pl
from jax.experimental.pallas import tpu as pltpu
from jax.experimental.pallas import tpu_sc as plsc

D_MODEL = 64
SCALE = 8.0

_NC = 2
_NS = 16
_NW = _NC * _NS


@functools.lru_cache(maxsize=None)
def _make_transpose(V: int, D: int):
    NB_FULL = V // 128
    TAIL = V - NB_FULL * 128
    KMAX = (NB_FULL + _NW - 1) // _NW
    tail_w = NB_FULL % _NW
    mesh = plsc.VectorSubcoreMesh(core_axis_name="c", subcore_axis_name="s")

    @functools.partial(
        pl.kernel,
        mesh=mesh,
        out_type=jax.ShapeDtypeStruct((V // 2, 2 * D), jnp.float32),
        scratch_types=[
            pltpu.VMEM((D, 128), jnp.float32),
            pltpu.VMEM((D, 128), jnp.float32),
            pltpu.VMEM((D, 128), jnp.float32),
            pltpu.VMEM((D, 128), jnp.float32),
            pltpu.VMEM((D, 128), jnp.float32),
            pltpu.SemaphoreType.DMA,
            pltpu.SemaphoreType.DMA,
            pltpu.SemaphoreType.DMA,
            pltpu.SemaphoreType.DMA,
        ],
        compiler_params=pltpu.CompilerParams(use_tc_tiling_on_sc=True, needs_layout_passes=False),
    )
    def k(tT, tail128, R, s0, s1, d0, d1, rot, si0, si1, so0, so1):
        w = lax.axis_index("s") * _NC + lax.axis_index("c")
        src = (s0, s1)
        dst = (d0, d1)
        semi = (si0, si1)
        semo = (so0, so1)
        iot = lax.iota(jnp.int32, 16)
        rows = [iot + 16 * jj for jj in range(4)]

        def blk_of(kk):
            return w + _NW * kk

        def start_in(b, kk):
            pltpu.make_async_copy(
                tT.at[:, pl.ds(blk_of(kk) * 128, 128)], src[b], semi[b]
            ).start()

        def wait_in(b):
            pltpu.make_async_copy(
                tT.at[:, pl.ds(0, 128)], src[b], semi[b]).wait()

        def start_out(b, kk):
            pltpu.make_async_copy(
                dst[b], R.at[pl.ds(blk_of(kk) * (128 // 2), 128 // 2)],
                semo[b]).start()

        def wait_out(b):
            pltpu.make_async_copy(
                dst[b], R.at[pl.ds(0, 128 // 2)], semo[b]).wait()

        def transpose_block(sref, dref, nrows):
            @plsc.parallel_loop(0, D, 1, unroll=2)
            def _(d):
                dv = jnp.full((16,), 0, jnp.int32) + d
                base = lax.bitwise_and(iot + dv, 127)
                for m in range(8):
                    colv = lax.bitwise_and(base + 16 * m, 127)
                    g = plsc.load_gather(sref, [dv, colv])
                    rot[d, pl.ds(16 * m, 16)] = g

            @plsc.parallel_loop(0, nrows, 1, unroll=2)
            def _(p):
                for half in range(2):
                    vspl = jnp.full((16,), 0, jnp.int32) + (2 * p + half)
                    for jj in range(D // 16):
                        colv = lax.bitwise_and(vspl - rows[jj], 127)
                        g = plsc.load_gather(rot, [rows[jj], colv])
                        dref[p, pl.ds((half * (D // 16) + jj) * 16, 16)] = (
                            g * SCALE)

        start_in(0, 0)

        def outer(o, carry):
            for b in (0, 1):
                kk = 2 * o + b
                ob = 1 - b

                @pl.when((kk + 1 <= KMAX - 1)
                         & (blk_of(kk + 1) < NB_FULL))
                def _():
                    start_in(ob, kk + 1)

                @pl.when((kk <= KMAX - 1) & (blk_of(kk) < NB_FULL))
                def _():
                    wait_in(b)

                    @pl.when(kk >= 2)
                    def _():
                        wait_out(b)

                    transpose_block(src[b], dst[b], 64)
                    start_out(b, kk)
            return carry

        lax.fori_loop(0, (KMAX + 1) // 2, outer, 0)
        wait_out(0)
        wait_out(1)

        if TAIL:
            @pl.when(w == tail_w)
            def _():
                pltpu.sync_copy(tail128, s0)
                transpose_block(s0, d0, TAIL // 2)
                pltpu.sync_copy(
                    d0.at[pl.ds(0, TAIL // 2)],
                    R.at[pl.ds(NB_FULL * (128 // 2), TAIL // 2)])

    return k


@functools.lru_cache(maxsize=None)
def _make_gather(S_LEN: int, B: int, V: int, D: int):
    NBB = B // 128
    assert NBB == _NW and S_LEN % 2 == 0
    mesh = plsc.VectorSubcoreMesh(core_axis_name="c", subcore_axis_name="s")

    @functools.partial(
        pl.kernel,
        mesh=mesh,
        out_type=jax.ShapeDtypeStruct((S_LEN, D, B), jnp.float32),
        scratch_types=[
            pltpu.VMEM((S_LEN, 128), jnp.int32),
            pltpu.VMEM((128,), jnp.int32),
            pltpu.VMEM((128,), jnp.int32),
            pltpu.VMEM((128,), jnp.int32),
            pltpu.VMEM((128,), jnp.int32),
            pltpu.VMEM((128, 2 * D), jnp.float32),
            pltpu.VMEM((128, 2 * D), jnp.float32),
            pltpu.VMEM((D, 128), jnp.float32),
            pltpu.VMEM((D, 128), jnp.float32),
            pltpu.VMEM((128, 128), jnp.float32),
            pltpu.SemaphoreType.DMA,
            pltpu.SemaphoreType.DMA,
            pltpu.SemaphoreType.DMA,
            pltpu.SemaphoreType.DMA,
        ],
        compiler_params=pltpu.CompilerParams(use_tc_tiling_on_sc=True, needs_layout_passes=False),
    )
    def k(xT, R, O, xall, ri0, ri1, pa0, pa1, g0, g1, sl0, sl1, grot,
          sg0, sg1, so0, so1):
        w = lax.axis_index("s") * _NC + lax.axis_index("c")
        ridx = (ri0, ri1)
        par = (pa0, pa1)
        G = (g0, g1)
        S = (sl0, sl1)
        semg = (sg0, sg1)
        semo = (so0, so1)
        iot = lax.iota(jnp.int32, 16)
        rowsc = [iot + 16 * c for c in range(8)]

        pltpu.sync_copy(xT.at[:, pl.ds(128 * w, 128)], xall)

        def prep(b, s):
            for m in range(8):
                v = xall[s, pl.ds(16 * m, 16)]
                ridx[b][pl.ds(16 * m, 16)] = lax.shift_right_logical(v, 1)
                par[b][pl.ds(16 * m, 16)] = lax.shift_left(
                    lax.bitwise_and(v, 1), 6)

        def start_gather(b):
            pltpu.make_async_copy(R.at[ridx[b]], G[b], semg[b]).start()

        def wait_gather(b):
            pltpu.make_async_copy(R.at[ridx[b]], G[b], semg[b]).wait()

        def start_out(b, s):
            pltpu.make_async_copy(
                S[b], O.at[s, :, pl.ds(128 * w, 128)], semo[b]).start()

        def wait_out(b):
            pltpu.make_async_copy(
                S[b], O.at[0, :, pl.ds(128 * w, 128)], semo[b]).wait()

        def transpose_unit(b):
            @plsc.parallel_loop(0, 128, 1, unroll=2)
            def _(r):
                rv = jnp.full((16,), 0, jnp.int32) + r
                base = lax.bitwise_and(iot + rv, 127)
                for m in range(8):
                    colv = lax.bitwise_and(base + 16 * m, 127)
                    g = plsc.load_gather(G[b], [rv, colv])
                    grot[r, pl.ds(16 * m, 16)] = g

            @plsc.parallel_loop(0, D, 1, unroll=2)
            def _(d):
                dv = jnp.full((16,), 0, jnp.int32) + d
                for c in range(8):
                    parc = par[b][pl.ds(16 * c, 16)]
                    colv = lax.bitwise_and(parc + dv - rowsc[c], 127)
                    g = plsc.load_gather(grot, [rowsc[c], colv])
                    S[b][d, pl.ds(16 * c, 16)] = g

        prep(0, 0)
        start_gather(0)

        def outer(o, carry):
            for b in (0, 1):
                s = 2 * o + b
                ob = 1 - b

                @pl.when(s + 1 <= S_LEN - 1)
                def _():
                    prep(ob, s + 1)
                    start_gather(ob)

                wait_gather(b)

                @pl.when(s >= 2)
                def _():
                    wait_out(b)

                transpose_unit(b)
                start_out(b, s)
            return carry

        lax.fori_loop(0, S_LEN // 2, outer, 0)
        wait_out(0)
        wait_out(1)

    return k


def kernel(x, table):
    B, S_LEN = x.shape
    V, D = table.shape
    tT = table.T
    tail = V % 128
    tail128 = jnp.pad(table[V - tail:], ((0, 128 - tail), (0, 0))).T
    R = _make_transpose(V, D)(tT, tail128)
    xT = x.astype(jnp.int32).T
    O = _make_gather(S_LEN, B, V, D)(xT, R)
    return jnp.transpose(O, (2, 0, 1))


# speedup vs baseline: 1.0454x; 1.0454x over previous
import functools

import jax
import jax.numpy as jnp
from jax import lax
from jax.experimental import pallas as pl
from jax.experimental.pallas import tpu as pltpu
from jax.experimental.pallas import tpu_sc as plsc

D_MODEL = 64
SCALE = 8.0

_NC = 2
_NS = 16
_NW = _NC * _NS


@functools.lru_cache(maxsize=None)
def _make_transpose(V: int, D: int):
    NB_FULL = V // 128
    TAIL = V - NB_FULL * 128
    KMAX = (NB_FULL + _NW - 1) // _NW
    tail_w = NB_FULL % _NW
    mesh = plsc.VectorSubcoreMesh(core_axis_name="c", subcore_axis_name="s")

    @functools.partial(
        pl.kernel,
        mesh=mesh,
        out_type=jax.ShapeDtypeStruct((V // 2, 2 * D), jnp.float32),
        scratch_types=[
            pltpu.VMEM((D, 128), jnp.float32),
            pltpu.VMEM((D, 128), jnp.float32),
            pltpu.VMEM((D, 128), jnp.float32),
            pltpu.VMEM((D, 128), jnp.float32),
            pltpu.VMEM((D, 128), jnp.float32),
            pltpu.SemaphoreType.DMA,
            pltpu.SemaphoreType.DMA,
            pltpu.SemaphoreType.DMA,
            pltpu.SemaphoreType.DMA,
        ],
        compiler_params=pltpu.CompilerParams(use_tc_tiling_on_sc=True, needs_layout_passes=False),
    )
    def k(tT, tail128, R, s0, s1, d0, d1, rot, si0, si1, so0, so1):
        w = lax.axis_index("s") * _NC + lax.axis_index("c")
        src = (s0, s1)
        dst = (d0, d1)
        semi = (si0, si1)
        semo = (so0, so1)
        iot = lax.iota(jnp.int32, 16)
        rows = [iot + 16 * jj for jj in range(4)]

        def blk_of(kk):
            return w + _NW * kk

        def start_in(b, kk):
            pltpu.make_async_copy(
                tT.at[:, pl.ds(blk_of(kk) * 128, 128)], src[b], semi[b]
            ).start()

        def wait_in(b):
            pltpu.make_async_copy(
                tT.at[:, pl.ds(0, 128)], src[b], semi[b]).wait()

        def start_out(b, kk):
            pltpu.make_async_copy(
                dst[b], R.at[pl.ds(blk_of(kk) * (128 // 2), 128 // 2)],
                semo[b]).start()

        def wait_out(b):
            pltpu.make_async_copy(
                dst[b], R.at[pl.ds(0, 128 // 2)], semo[b]).wait()

        def transpose_block(sref, dref, nrows):
            @plsc.parallel_loop(0, D, 1, unroll=2)
            def _(d):
                dv = jnp.full((16,), 0, jnp.int32) + d
                base = lax.bitwise_and(iot + dv, 127)
                for m in range(8):
                    colv = lax.bitwise_and(base + 16 * m, 127)
                    g = plsc.load_gather(sref, [dv, colv])
                    rot[d, pl.ds(16 * m, 16)] = g

            @plsc.parallel_loop(0, nrows, 1, unroll=2)
            def _(p):
                for half in range(2):
                    vspl = jnp.full((16,), 0, jnp.int32) + (2 * p + half)
                    for jj in range(D // 16):
                        colv = lax.bitwise_and(vspl - rows[jj], 127)
                        g = plsc.load_gather(rot, [rows[jj], colv])
                        dref[p, pl.ds((half * (D // 16) + jj) * 16, 16)] = (
                            g * SCALE)

        start_in(0, 0)

        def outer(o, carry):
            for b in (0, 1):
                kk = 2 * o + b
                ob = 1 - b

                @pl.when((kk + 1 <= KMAX - 1)
                         & (blk_of(kk + 1) < NB_FULL))
                def _():
                    start_in(ob, kk + 1)

                @pl.when((kk <= KMAX - 1) & (blk_of(kk) < NB_FULL))
                def _():
                    wait_in(b)

                    @pl.when(kk >= 2)
                    def _():
                        wait_out(b)

                    transpose_block(src[b], dst[b], 64)
                    start_out(b, kk)
            return carry

        lax.fori_loop(0, (KMAX + 1) // 2, outer, 0)
        wait_out(0)
        wait_out(1)

        if TAIL:
            @pl.when(w == tail_w)
            def _():
                pltpu.sync_copy(tail128, s0)
                transpose_block(s0, d0, TAIL // 2)
                pltpu.sync_copy(
                    d0.at[pl.ds(0, TAIL // 2)],
                    R.at[pl.ds(NB_FULL * (128 // 2), TAIL // 2)])

    return k


@functools.lru_cache(maxsize=None)
def _make_gather(S_LEN: int, B: int, V: int, D: int):
    NBB = B // 128
    assert NBB == _NW and S_LEN % 2 == 0
    mesh = plsc.VectorSubcoreMesh(core_axis_name="c", subcore_axis_name="s")

    @functools.partial(
        pl.kernel,
        mesh=mesh,
        out_type=jax.ShapeDtypeStruct((S_LEN, D, B), jnp.float32),
        scratch_types=[
            pltpu.VMEM((S_LEN, 128), jnp.int32),
            pltpu.VMEM((128,), jnp.int32),
            pltpu.VMEM((128,), jnp.int32),
            pltpu.VMEM((144,), jnp.int32),
            pltpu.VMEM((144,), jnp.int32),
            pltpu.VMEM((128, 2 * D), jnp.float32),
            pltpu.VMEM((128, 2 * D), jnp.float32),
            pltpu.VMEM((D, 128), jnp.float32),
            pltpu.VMEM((D, 128), jnp.float32),
            pltpu.VMEM((128, D), jnp.float32),
            pltpu.SemaphoreType.DMA,
            pltpu.SemaphoreType.DMA,
            pltpu.SemaphoreType.DMA,
            pltpu.SemaphoreType.DMA,
        ],
        compiler_params=pltpu.CompilerParams(use_tc_tiling_on_sc=True, needs_layout_passes=False),
    )
    def k(xT, R, O, xall, ri0, ri1, pa0, pa1, g0, g1, sl0, sl1, grot,
          sg0, sg1, so0, so1):
        w = lax.axis_index("s") * _NC + lax.axis_index("c")
        ridx = (ri0, ri1)
        par = (pa0, pa1)
        G = (g0, g1)
        S = (sl0, sl1)
        semg = (sg0, sg1)
        semo = (so0, so1)
        iot = lax.iota(jnp.int32, 16)
        rowsc = [iot + 16 * c for c in range(8)]

        pltpu.sync_copy(xT.at[:, pl.ds(128 * w, 128)], xall)

        def prep(b, s):
            for m in range(8):
                v = xall[s, pl.ds(16 * m, 16)]
                ridx[b][pl.ds(16 * m, 16)] = lax.shift_right_logical(v, 1)
                par[b][pl.ds(16 * m, 16)] = lax.shift_left(
                    lax.bitwise_and(v, 1), 6)

        def start_gather(b):
            pltpu.make_async_copy(R.at[ridx[b]], G[b], semg[b]).start()

        def wait_gather(b):
            pltpu.make_async_copy(R.at[ridx[b]], G[b], semg[b]).wait()

        def start_out(b, s):
            pltpu.make_async_copy(
                S[b], O.at[s, :, pl.ds(128 * w, 128)], semo[b]).start()

        def wait_out(b):
            pltpu.make_async_copy(
                S[b], O.at[0, :, pl.ds(128 * w, 128)], semo[b]).wait()

        def transpose_unit(b):
            @plsc.parallel_loop(0, 128, 1, unroll=4)
            def _(r):
                pr = par[b][pl.ds(r, 16)][0]
                rv = jnp.full((16,), 0, jnp.int32) + r
                pv = jnp.full((16,), 0, jnp.int32) + pr
                base = lax.bitwise_and(iot + rv, D - 1)
                for m in range(D // 16):
                    colv = lax.bitwise_and(base + 16 * m, D - 1) + pv
                    g = plsc.load_gather(G[b], [rv, colv])
                    grot[r, pl.ds(16 * m, 16)] = g

            @plsc.parallel_loop(0, D, 1, unroll=4)
            def _(d):
                dv = jnp.full((16,), 0, jnp.int32) + d
                for c in range(8):
                    colv = lax.bitwise_and(dv - rowsc[c], D - 1)
                    g = plsc.load_gather(grot, [rowsc[c], colv])
                    S[b][d, pl.ds(16 * c, 16)] = g

        prep(0, 0)
        start_gather(0)

        def outer(o, carry):
            for b in (0, 1):
                s = 2 * o + b
                ob = 1 - b

                @pl.when(s + 1 <= S_LEN - 1)
                def _():
                    prep(ob, s + 1)
                    start_gather(ob)

                wait_gather(b)

                @pl.when(s >= 2)
                def _():
                    wait_out(b)

                transpose_unit(b)
                start_out(b, s)
            return carry

        lax.fori_loop(0, S_LEN // 2, outer, 0)
        wait_out(0)
        wait_out(1)

    return k


def kernel(x, table):
    B, S_LEN = x.shape
    V, D = table.shape
    tT = table.T
    tail = V % 128
    tail128 = jnp.pad(table[V - tail:], ((0, 128 - tail), (0, 0))).T
    R = _make_transpose(V, D)(tT, tail128)
    xT = x.astype(jnp.int32).T
    O = _make_gather(S_LEN, B, V, D)(xT, R)
    return jnp.transpose(O, (2, 0, 1))


# speedup vs baseline: 1.0490x; 1.0034x over previous
import functools

import jax
import jax.numpy as jnp
from jax import lax
from jax.experimental import pallas as pl
from jax.experimental.pallas import tpu as pltpu
from jax.experimental.pallas import tpu_sc as plsc

D_MODEL = 64
SCALE = 8.0

_NC = 2
_NS = 16
_NW = _NC * _NS


@functools.lru_cache(maxsize=None)
def _make_transpose(V: int, D: int):
    NB_FULL = V // 128
    TAIL = V - NB_FULL * 128
    KMAX = (NB_FULL + _NW - 1) // _NW
    tail_w = NB_FULL % _NW
    mesh = plsc.VectorSubcoreMesh(core_axis_name="c", subcore_axis_name="s")

    @functools.partial(
        pl.kernel,
        mesh=mesh,
        out_type=jax.ShapeDtypeStruct((V // 2, 2 * D), jnp.float32),
        scratch_types=[
            pltpu.VMEM((D, 128), jnp.float32),
            pltpu.VMEM((D, 128), jnp.float32),
            pltpu.VMEM((D, 128), jnp.float32),
            pltpu.VMEM((D, 128), jnp.float32),
            pltpu.VMEM((D, 128), jnp.float32),
            pltpu.SemaphoreType.DMA,
            pltpu.SemaphoreType.DMA,
            pltpu.SemaphoreType.DMA,
            pltpu.SemaphoreType.DMA,
        ],
        compiler_params=pltpu.CompilerParams(use_tc_tiling_on_sc=True, needs_layout_passes=False),
    )
    def k(tT, tail128, R, s0, s1, d0, d1, rot, si0, si1, so0, so1):
        w = lax.axis_index("s") * _NC + lax.axis_index("c")
        src = (s0, s1)
        dst = (d0, d1)
        semi = (si0, si1)
        semo = (so0, so1)
        iot = lax.iota(jnp.int32, 16)
        rows = [iot + 16 * jj for jj in range(4)]

        def blk_of(kk):
            return w + _NW * kk

        def start_in(b, kk):
            pltpu.make_async_copy(
                tT.at[:, pl.ds(blk_of(kk) * 128, 128)], src[b], semi[b]
            ).start()

        def wait_in(b):
            pltpu.make_async_copy(
                tT.at[:, pl.ds(0, 128)], src[b], semi[b]).wait()

        def start_out(b, kk):
            pltpu.make_async_copy(
                dst[b], R.at[pl.ds(blk_of(kk) * (128 // 2), 128 // 2)],
                semo[b]).start()

        def wait_out(b):
            pltpu.make_async_copy(
                dst[b], R.at[pl.ds(0, 128 // 2)], semo[b]).wait()

        def transpose_block(sref, dref, nrows):
            @plsc.parallel_loop(0, D, 1, unroll=4)
            def _(d):
                dv = jnp.full((16,), 0, jnp.int32) + d
                base = lax.bitwise_and(iot + dv, 127)
                for m in range(8):
                    colv = lax.bitwise_and(base + 16 * m, 127)
                    g = plsc.load_gather(sref, [dv, colv])
                    rot[d, pl.ds(16 * m, 16)] = g

            @plsc.parallel_loop(0, nrows, 1, unroll=4)
            def _(p):
                for half in range(2):
                    vspl = jnp.full((16,), 0, jnp.int32) + (2 * p + half)
                    for jj in range(D // 16):
                        colv = lax.bitwise_and(vspl - rows[jj], 127)
                        g = plsc.load_gather(rot, [rows[jj], colv])
                        dref[p, pl.ds((half * (D // 16) + jj) * 16, 16)] = (
                            g * SCALE)

        start_in(0, 0)

        def outer(o, carry):
            for b in (0, 1):
                kk = 2 * o + b
                ob = 1 - b

                @pl.when((kk + 1 <= KMAX - 1)
                         & (blk_of(kk + 1) < NB_FULL))
                def _():
                    start_in(ob, kk + 1)

                @pl.when((kk <= KMAX - 1) & (blk_of(kk) < NB_FULL))
                def _():
                    wait_in(b)

                    @pl.when(kk >= 2)
                    def _():
                        wait_out(b)

                    transpose_block(src[b], dst[b], 64)
                    start_out(b, kk)
            return carry

        lax.fori_loop(0, (KMAX + 1) // 2, outer, 0)
        wait_out(0)
        wait_out(1)

        if TAIL:
            @pl.when(w == tail_w)
            def _():
                pltpu.sync_copy(tail128, s0)
                transpose_block(s0, d0, TAIL // 2)
                pltpu.sync_copy(
                    d0.at[pl.ds(0, TAIL // 2)],
                    R.at[pl.ds(NB_FULL * (128 // 2), TAIL // 2)])

    return k


@functools.lru_cache(maxsize=None)
def _make_gather(S_LEN: int, B: int, V: int, D: int):
    NBB = B // 128
    assert NBB == _NW and S_LEN % 2 == 0
    mesh = plsc.VectorSubcoreMesh(core_axis_name="c", subcore_axis_name="s")

    @functools.partial(
        pl.kernel,
        mesh=mesh,
        out_type=jax.ShapeDtypeStruct((S_LEN, D, B), jnp.float32),
        scratch_types=[
            pltpu.VMEM((S_LEN, 128), jnp.int32),
            pltpu.VMEM((128,), jnp.int32),
            pltpu.VMEM((128,), jnp.int32),
            pltpu.VMEM((144,), jnp.int32),
            pltpu.VMEM((144,), jnp.int32),
            pltpu.VMEM((128, 2 * D), jnp.float32),
            pltpu.VMEM((128, 2 * D), jnp.float32),
            pltpu.VMEM((D, 128), jnp.float32),
            pltpu.VMEM((D, 128), jnp.float32),
            pltpu.VMEM((128, D), jnp.float32),
            pltpu.SemaphoreType.DMA,
            pltpu.SemaphoreType.DMA,
            pltpu.SemaphoreType.DMA,
            pltpu.SemaphoreType.DMA,
        ],
        compiler_params=pltpu.CompilerParams(use_tc_tiling_on_sc=True, needs_layout_passes=False),
    )
    def k(xT, R, O, xall, ri0, ri1, pa0, pa1, g0, g1, sl0, sl1, grot,
          sg0, sg1, so0, so1):
        w = lax.axis_index("s") * _NC + lax.axis_index("c")
        ridx = (ri0, ri1)
        par = (pa0, pa1)
        G = (g0, g1)
        S = (sl0, sl1)
        semg = (sg0, sg1)
        semo = (so0, so1)
        iot = lax.iota(jnp.int32, 16)
        rowsc = [iot + 16 * c for c in range(8)]

        pltpu.sync_copy(xT.at[:, pl.ds(128 * w, 128)], xall)

        def prep(b, s):
            for m in range(8):
                v = xall[s, pl.ds(16 * m, 16)]
                ridx[b][pl.ds(16 * m, 16)] = lax.shift_right_logical(v, 1)
                par[b][pl.ds(16 * m, 16)] = lax.shift_left(
                    lax.bitwise_and(v, 1), 6)

        def start_gather(b):
            pltpu.make_async_copy(R.at[ridx[b]], G[b], semg[b]).start()

        def wait_gather(b):
            pltpu.make_async_copy(R.at[ridx[b]], G[b], semg[b]).wait()

        def start_out(b, s):
            pltpu.make_async_copy(
                S[b], O.at[s, :, pl.ds(128 * w, 128)], semo[b]).start()

        def wait_out(b):
            pltpu.make_async_copy(
                S[b], O.at[0, :, pl.ds(128 * w, 128)], semo[b]).wait()

        def transpose_unit(b):
            @plsc.parallel_loop(0, 128, 1, unroll=4)
            def _(r):
                pr = par[b][pl.ds(r, 16)][0]
                rv = jnp.full((16,), 0, jnp.int32) + r
                pv = jnp.full((16,), 0, jnp.int32) + pr
                base = lax.bitwise_and(iot + rv, D - 1)
                for m in range(D // 16):
                    colv = lax.bitwise_and(base + 16 * m, D - 1) + pv
                    g = plsc.load_gather(G[b], [rv, colv])
                    grot[r, pl.ds(16 * m, 16)] = g

            @plsc.parallel_loop(0, D, 1, unroll=4)
            def _(d):
                dv = jnp.full((16,), 0, jnp.int32) + d
                for c in range(8):
                    colv = lax.bitwise_and(dv - rowsc[c], D - 1)
                    g = plsc.load_gather(grot, [rowsc[c], colv])
                    S[b][d, pl.ds(16 * c, 16)] = g

        prep(0, 0)
        start_gather(0)

        def outer(o, carry):
            for b in (0, 1):
                s = 2 * o + b
                ob = 1 - b

                @pl.when(s + 1 <= S_LEN - 1)
                def _():
                    prep(ob, s + 1)
                    start_gather(ob)

                wait_gather(b)

                @pl.when(s >= 2)
                def _():
                    wait_out(b)

                transpose_unit(b)
                start_out(b, s)
            return carry

        lax.fori_loop(0, S_LEN // 2, outer, 0)
        wait_out(0)
        wait_out(1)

    return k


def kernel(x, table):
    B, S_LEN = x.shape
    V, D = table.shape
    tT = table.T
    tail = V % 128
    tail128 = jnp.pad(table[V - tail:], ((0, 128 - tail), (0, 0))).T
    R = _make_transpose(V, D)(tT, tail128)
    xT = x.astype(jnp.int32).T
    O = _make_gather(S_LEN, B, V, D)(xT, R)
    return jnp.transpose(O, (2, 0, 1))


# speedup vs baseline: 1.2388x; 1.1809x over previous
import functools

import jax
import jax.numpy as jnp
from jax import lax
from jax.experimental import pallas as pl
from jax.experimental.pallas import tpu as pltpu
from jax.experimental.pallas import tpu_sc as plsc

D_MODEL = 64
SCALE = 8.0

_NC = 2
_NS = 16
_NW = _NC * _NS


@functools.lru_cache(maxsize=None)
def _make_transpose(V: int, D: int):
    NB_FULL = V // 128
    TAIL = V - NB_FULL * 128
    KMAX = (NB_FULL + _NW - 1) // _NW
    tail_w = NB_FULL % _NW
    mesh = plsc.VectorSubcoreMesh(core_axis_name="c", subcore_axis_name="s")

    @functools.partial(
        pl.kernel,
        mesh=mesh,
        out_type=jax.ShapeDtypeStruct((V // 2, 2 * D), jnp.float32),
        scratch_types=[
            pltpu.VMEM((D, 128), jnp.float32),
            pltpu.VMEM((D, 128), jnp.float32),
            pltpu.VMEM((D, 128), jnp.float32),
            pltpu.VMEM((D, 128), jnp.float32),
            pltpu.VMEM((D, 128), jnp.float32),
            pltpu.SemaphoreType.DMA,
            pltpu.SemaphoreType.DMA,
            pltpu.SemaphoreType.DMA,
            pltpu.SemaphoreType.DMA,
        ],
        compiler_params=pltpu.CompilerParams(use_tc_tiling_on_sc=True, needs_layout_passes=False),
    )
    def k(tT, tail128, R, s0, s1, d0, d1, rot, si0, si1, so0, so1):
        w = lax.axis_index("s") * _NC + lax.axis_index("c")
        src = (s0, s1)
        dst = (d0, d1)
        semi = (si0, si1)
        semo = (so0, so1)
        iot = lax.iota(jnp.int32, 16)
        rows = [iot + 16 * jj for jj in range(4)]

        def blk_of(kk):
            return w + _NW * kk

        def start_in(b, kk):
            pltpu.make_async_copy(
                tT.at[:, pl.ds(blk_of(kk) * 128, 128)], src[b], semi[b]
            ).start()

        def wait_in(b):
            pltpu.make_async_copy(
                tT.at[:, pl.ds(0, 128)], src[b], semi[b]).wait()

        def start_out(b, kk):
            pltpu.make_async_copy(
                dst[b], R.at[pl.ds(blk_of(kk) * (128 // 2), 128 // 2)],
                semo[b]).start()

        def wait_out(b):
            pltpu.make_async_copy(
                dst[b], R.at[pl.ds(0, 128 // 2)], semo[b]).wait()

        def transpose_block(sref, dref, nrows):
            @plsc.parallel_loop(0, D, 1, unroll=4)
            def _(d):
                dv = jnp.full((16,), 0, jnp.int32) + d
                base = lax.bitwise_and(iot + dv, 127)
                for m in range(8):
                    colv = lax.bitwise_and(base + 16 * m, 127)
                    g = plsc.load_gather(sref, [dv, colv])
                    rot[d, pl.ds(16 * m, 16)] = g

            @plsc.parallel_loop(0, nrows, 1, unroll=4)
            def _(p):
                for half in range(2):
                    vspl = jnp.full((16,), 0, jnp.int32) + (2 * p + half)
                    for jj in range(D // 16):
                        colv = lax.bitwise_and(vspl - rows[jj], 127)
                        g = plsc.load_gather(rot, [rows[jj], colv])
                        dref[p, pl.ds((half * (D // 16) + jj) * 16, 16)] = (
                            g * SCALE)

        start_in(0, 0)

        def outer(o, carry):
            for b in (0, 1):
                kk = 2 * o + b
                ob = 1 - b

                @pl.when((kk + 1 <= KMAX - 1)
                         & (blk_of(kk + 1) < NB_FULL))
                def _():
                    start_in(ob, kk + 1)

                @pl.when((kk <= KMAX - 1) & (blk_of(kk) < NB_FULL))
                def _():
                    wait_in(b)

                    @pl.when(kk >= 2)
                    def _():
                        wait_out(b)

                    transpose_block(src[b], dst[b], 64)
                    start_out(b, kk)
            return carry

        lax.fori_loop(0, (KMAX + 1) // 2, outer, 0)
        wait_out(0)
        wait_out(1)

        if TAIL:
            @pl.when(w == tail_w)
            def _():
                pltpu.sync_copy(tail128, s0)
                transpose_block(s0, d0, TAIL // 2)
                pltpu.sync_copy(
                    d0.at[pl.ds(0, TAIL // 2)],
                    R.at[pl.ds(NB_FULL * (128 // 2), TAIL // 2)])

    return k


@functools.lru_cache(maxsize=None)
def _make_gather(S_LEN: int, B: int, V: int, D: int):
    NBB = B // 128
    assert NBB == _NW and S_LEN % 2 == 0 and D == 64
    mesh = plsc.VectorSubcoreMesh(core_axis_name="c", subcore_axis_name="s")

    @functools.partial(
        pl.kernel,
        mesh=mesh,
        out_type=jax.ShapeDtypeStruct((S_LEN, D // 8, _NW, 8 * 128),
                                      jnp.float32),
        scratch_types=[
            pltpu.VMEM((S_LEN, 128), jnp.int32),
            pltpu.VMEM((128, D), jnp.float32),
            pltpu.VMEM((128, D), jnp.float32),
            pltpu.VMEM((D // 8, 8 * 128), jnp.float32),
            pltpu.VMEM((D // 8, 8 * 128), jnp.float32),
            pltpu.VMEM((128, D), jnp.float32),
            pltpu.SemaphoreType.DMA,
            pltpu.SemaphoreType.DMA,
            pltpu.SemaphoreType.DMA,
            pltpu.SemaphoreType.DMA,
            pltpu.SemaphoreType.DMA,
        ],
        compiler_params=pltpu.CompilerParams(use_tc_tiling_on_sc=False, needs_layout_passes=False),
    )
    def k(xT, R, O, xall, g0, g1, sl0, sl1, grot, sx, sg0, sg1, so0, so1):
        w = lax.axis_index("s") * _NC + lax.axis_index("c")
        G = (g0, g1)
        S = (sl0, sl1)
        semg = (sg0, sg1)
        semo = (so0, so1)
        iot = lax.iota(jnp.int32, 16)
        rowsc = [iot + 16 * c for c in range(8)]

        pltpu.make_async_copy(
            xT.at[:, pl.ds(128 * w, 128)], xall, sx).start()
        pltpu.make_async_copy(
            xT.at[:, pl.ds(0, 128)], xall, sx).wait()

        def start_gather(b, s):
            pltpu.make_async_copy(R.at[xall.at[s]], G[b], semg[b]).start()

        def wait_gather(b):
            pltpu.make_async_copy(R.at[xall.at[0]], G[b], semg[b]).wait()

        def start_out(b, s):
            pltpu.make_async_copy(
                S[b], O.at[s, :, w, :], semo[b]).start()

        def wait_out(b):
            pltpu.make_async_copy(
                S[b], O.at[0, :, w, :], semo[b]).wait()

        def transpose_unit(b):
            @plsc.parallel_loop(0, 128, 1, unroll=4)
            def _(r):
                rv = jnp.full((16,), 0, jnp.int32) + r
                base = lax.bitwise_and(iot + rv, D - 1)
                for m in range(D // 16):
                    colv = lax.bitwise_and(base + 16 * m, D - 1)
                    g = plsc.load_gather(G[b], [rv, colv])
                    grot[r, pl.ds(16 * m, 16)] = g

            @plsc.parallel_loop(0, D, 1, unroll=4)
            def _(d):
                dv = jnp.full((16,), 0, jnp.int32) + d
                dhi = d // 8
                dlo = d - dhi * 8
                for c in range(8):
                    colv = lax.bitwise_and(dv - rowsc[c], D - 1)
                    g = plsc.load_gather(grot, [rowsc[c], colv])
                    S[b][dhi, pl.ds(dlo * 128 + 16 * c, 16)] = g

        start_gather(0, 0)

        def outer(o, carry):
            for b in (0, 1):
                s = 2 * o + b
                ob = 1 - b

                @pl.when(s + 1 <= S_LEN - 1)
                def _():
                    start_gather(ob, s + 1)

                wait_gather(b)

                @pl.when(s >= 2)
                def _():
                    wait_out(b)

                transpose_unit(b)
                start_out(b, s)
            return carry

        lax.fori_loop(0, S_LEN // 2, outer, 0)
        wait_out(0)
        wait_out(1)

    return k


def kernel(x, table):
    B, S_LEN = x.shape
    V, D = table.shape
    tT = table.T
    tail = V % 128
    tail128 = jnp.pad(table[V - tail:], ((0, 128 - tail), (0, 0))).T
    R = _make_transpose(V, D)(tT, tail128)
    xT = x.astype(jnp.int32).T
    O = _make_gather(S_LEN, B, V, D)(xT, R.reshape(V, D))
    o5 = O.reshape(S_LEN, D // 8, _NW, 8, 128)
    return o5.transpose(2, 4, 0, 1, 3).reshape(B, S_LEN, D)


# speedup vs baseline: 1.4030x; 1.1325x over previous
import functools

import jax
import jax.numpy as jnp
from jax import lax
from jax.experimental import pallas as pl
from jax.experimental.pallas import tpu as pltpu
from jax.experimental.pallas import tpu_sc as plsc

D_MODEL = 64
SCALE = 8.0

_NC = 2
_NS = 16
_NW = _NC * _NS


@functools.lru_cache(maxsize=None)
def _make_transpose(V: int, D: int):
    NB_FULL = V // 128
    TAIL = V - NB_FULL * 128
    KMAX = (NB_FULL + _NW - 1) // _NW
    tail_w = NB_FULL % _NW
    mesh = plsc.VectorSubcoreMesh(core_axis_name="c", subcore_axis_name="s")

    @functools.partial(
        pl.kernel,
        mesh=mesh,
        out_type=jax.ShapeDtypeStruct((V // 2, 2 * D), jnp.float32),
        scratch_types=[
            pltpu.VMEM((D, 128), jnp.float32),
            pltpu.VMEM((D, 128), jnp.float32),
            pltpu.VMEM((D, 128), jnp.float32),
            pltpu.VMEM((D, 128), jnp.float32),
            pltpu.VMEM((D, 128), jnp.float32),
            pltpu.VMEM((D, 128), jnp.float32),
            pltpu.VMEM((D, 128), jnp.float32),
            pltpu.SemaphoreType.DMA,
            pltpu.SemaphoreType.DMA,
            pltpu.SemaphoreType.DMA,
            pltpu.SemaphoreType.DMA,
            pltpu.SemaphoreType.DMA,
            pltpu.SemaphoreType.DMA,
        ],
        compiler_params=pltpu.CompilerParams(use_tc_tiling_on_sc=True, needs_layout_passes=False),
    )
    def k(tT, tail128, R, s0, s1, s2, d0, d1, d2, rot, si0, si1, si2,
          so0, so1, so2):
        w = lax.axis_index("s") * _NC + lax.axis_index("c")
        src = (s0, s1, s2)
        dst = (d0, d1, d2)
        semi = (si0, si1, si2)
        semo = (so0, so1, so2)
        iot = lax.iota(jnp.int32, 16)
        rows = [iot + 16 * jj for jj in range(4)]

        def blk_of(kk):
            return w + _NW * kk

        def start_in(b, kk):
            pltpu.make_async_copy(
                tT.at[:, pl.ds(blk_of(kk) * 128, 128)], src[b], semi[b]
            ).start()

        def wait_in(b):
            pltpu.make_async_copy(
                tT.at[:, pl.ds(0, 128)], src[b], semi[b]).wait()

        def start_out(b, kk):
            pltpu.make_async_copy(
                dst[b], R.at[pl.ds(blk_of(kk) * (128 // 2), 128 // 2)],
                semo[b]).start()

        def wait_out(b):
            pltpu.make_async_copy(
                dst[b], R.at[pl.ds(0, 128 // 2)], semo[b]).wait()

        def transpose_block(sref, dref, nrows):
            @plsc.parallel_loop(0, D, 1, unroll=4)
            def _(d):
                dv = jnp.full((16,), 0, jnp.int32) + d
                base = lax.bitwise_and(iot + dv, 127)
                for m in range(8):
                    colv = lax.bitwise_and(base + 16 * m, 127)
                    g = plsc.load_gather(sref, [dv, colv])
                    rot[d, pl.ds(16 * m, 16)] = g

            @plsc.parallel_loop(0, nrows, 1, unroll=4)
            def _(p):
                for half in range(2):
                    vspl = jnp.full((16,), 0, jnp.int32) + (2 * p + half)
                    for jj in range(D // 16):
                        colv = lax.bitwise_and(vspl - rows[jj], 127)
                        g = plsc.load_gather(rot, [rows[jj], colv])
                        dref[p, pl.ds((half * (D // 16) + jj) * 16, 16)] = (
                            g * SCALE)

        start_in(0, 0)
        start_in(1, 1)

        def outer(o, carry):
            for b in (0, 1, 2):
                kk = 3 * o + b
                b2 = (b + 2) % 3

                @pl.when((kk + 2 <= KMAX - 1)
                         & (blk_of(kk + 2) < NB_FULL))
                def _():
                    start_in(b2, kk + 2)

                @pl.when((kk <= KMAX - 1) & (blk_of(kk) < NB_FULL))
                def _():
                    wait_in(b)

                    @pl.when(kk >= 3)
                    def _():
                        wait_out(b)

                    transpose_block(src[b], dst[b], 64)
                    start_out(b, kk)
            return carry

        lax.fori_loop(0, (KMAX + 2) // 3, outer, 0)
        wait_out(0)
        wait_out(1)
        wait_out(2)

        if TAIL:
            @pl.when(w == tail_w)
            def _():
                pltpu.sync_copy(tail128, s0)
                transpose_block(s0, d0, TAIL // 2)
                pltpu.sync_copy(
                    d0.at[pl.ds(0, TAIL // 2)],
                    R.at[pl.ds(NB_FULL * (128 // 2), TAIL // 2)])

    return k


@functools.lru_cache(maxsize=None)
def _make_gather(S_LEN: int, B: int, V: int, D: int):
    NBB = B // 128
    assert NBB == _NW and S_LEN % 2 == 0 and D == 64
    mesh = plsc.VectorSubcoreMesh(core_axis_name="c", subcore_axis_name="s")

    @functools.partial(
        pl.kernel,
        mesh=mesh,
        out_type=jax.ShapeDtypeStruct((S_LEN, D // 8, _NW, 8 * 128),
                                      jnp.float32),
        scratch_types=[
            pltpu.VMEM((S_LEN, 128), jnp.int32),
            pltpu.VMEM((128, D), jnp.float32),
            pltpu.VMEM((128, D), jnp.float32),
            pltpu.VMEM((D // 8, 8 * 128), jnp.float32),
            pltpu.VMEM((D // 8, 8 * 128), jnp.float32),
            pltpu.VMEM((128, D), jnp.float32),
            pltpu.SemaphoreType.DMA,
            pltpu.SemaphoreType.DMA,
            pltpu.SemaphoreType.DMA,
            pltpu.SemaphoreType.DMA,
            pltpu.SemaphoreType.DMA,
        ],
        compiler_params=pltpu.CompilerParams(use_tc_tiling_on_sc=False, needs_layout_passes=False),
    )
    def k(xT, R, O, xall, g0, g1, sl0, sl1, grot, sx, sg0, sg1, so0, so1):
        w = lax.axis_index("s") * _NC + lax.axis_index("c")
        G = (g0, g1)
        S = (sl0, sl1)
        semg = (sg0, sg1)
        semo = (so0, so1)
        iot = lax.iota(jnp.int32, 16)
        rowsc = [iot + 16 * c for c in range(8)]

        pltpu.make_async_copy(
            xT.at[:, pl.ds(128 * w, 128)], xall, sx).start()
        pltpu.make_async_copy(
            xT.at[:, pl.ds(0, 128)], xall, sx).wait()

        def start_gather(b, s):
            pltpu.make_async_copy(R.at[xall.at[s]], G[b], semg[b]).start()

        def wait_gather(b):
            pltpu.make_async_copy(R.at[xall.at[0]], G[b], semg[b]).wait()

        def start_out(b, s):
            pltpu.make_async_copy(
                S[b], O.at[s, :, w, :], semo[b]).start()

        def wait_out(b):
            pltpu.make_async_copy(
                S[b], O.at[0, :, w, :], semo[b]).wait()

        def transpose_unit(b):
            @plsc.parallel_loop(0, 128, 1, unroll=4)
            def _(r):
                rv = jnp.full((16,), 0, jnp.int32) + r
                base = lax.bitwise_and(iot + rv, D - 1)
                for m in range(D // 16):
                    colv = lax.bitwise_and(base + 16 * m, D - 1)
                    g = plsc.load_gather(G[b], [rv, colv])
                    grot[r, pl.ds(16 * m, 16)] = g

            @plsc.parallel_loop(0, D, 1, unroll=4)
            def _(d):
                dv = jnp.full((16,), 0, jnp.int32) + d
                dhi = d // 8
                dlo = d - dhi * 8
                for c in range(8):
                    colv = lax.bitwise_and(dv - rowsc[c], D - 1)
                    g = plsc.load_gather(grot, [rowsc[c], colv])
                    S[b][dhi, pl.ds(dlo * 128 + 16 * c, 16)] = g

        start_gather(0, 0)

        def outer(o, carry):
            for b in (0, 1):
                s = 2 * o + b
                ob = 1 - b

                @pl.when(s + 1 <= S_LEN - 1)
                def _():
                    start_gather(ob, s + 1)

                wait_gather(b)

                @pl.when(s >= 2)
                def _():
                    wait_out(b)

                transpose_unit(b)
                start_out(b, s)
            return carry

        lax.fori_loop(0, S_LEN // 2, outer, 0)
        wait_out(0)
        wait_out(1)

    return k


def kernel(x, table):
    B, S_LEN = x.shape
    V, D = table.shape
    tT = table.T
    tail = V % 128
    tail128 = jnp.pad(table[V - tail:], ((0, 128 - tail), (0, 0))).T
    R = _make_transpose(V, D)(tT, tail128)
    xT = x.astype(jnp.int32).T
    O = _make_gather(S_LEN, B, V, D)(xT, R.reshape(V, D))
    o5 = O.reshape(S_LEN, D // 8, _NW, 8, 128)
    return o5.transpose(2, 4, 0, 1, 3).reshape(B, S_LEN, D)


# speedup vs baseline: 1.5525x; 1.1066x over previous
import functools

import jax
import jax.numpy as jnp
from jax import lax
from jax.experimental import pallas as pl
from jax.experimental.pallas import tpu as pltpu
from jax.experimental.pallas import tpu_sc as plsc

D_MODEL = 64
SCALE = 8.0

_NC = 2
_NS = 16
_NW = _NC * _NS


@functools.lru_cache(maxsize=None)
def _make_transpose(V: int, D: int):
    NB_FULL = V // 128
    TAIL = V - NB_FULL * 128
    KMAX = (NB_FULL + _NW - 1) // _NW
    tail_w = NB_FULL % _NW
    mesh = plsc.VectorSubcoreMesh(core_axis_name="c", subcore_axis_name="s")

    @functools.partial(
        pl.kernel,
        mesh=mesh,
        out_type=jax.ShapeDtypeStruct((V // 2, 2 * D), jnp.float32),
        scratch_types=[
            pltpu.VMEM((D, 128), jnp.float32),
            pltpu.VMEM((D, 128), jnp.float32),
            pltpu.VMEM((D, 128), jnp.float32),
            pltpu.VMEM((D, 128), jnp.float32),
            pltpu.VMEM((D, 128), jnp.float32),
            pltpu.VMEM((D, 128), jnp.float32),
            pltpu.VMEM((D, 128), jnp.float32),
            pltpu.SemaphoreType.DMA,
            pltpu.SemaphoreType.DMA,
            pltpu.SemaphoreType.DMA,
            pltpu.SemaphoreType.DMA,
            pltpu.SemaphoreType.DMA,
            pltpu.SemaphoreType.DMA,
        ],
        compiler_params=pltpu.CompilerParams(use_tc_tiling_on_sc=True, needs_layout_passes=False),
    )
    def k(tT, tail128, R, s0, s1, s2, d0, d1, d2, rot, si0, si1, si2,
          so0, so1, so2):
        w = lax.axis_index("s") * _NC + lax.axis_index("c")
        src = (s0, s1, s2)
        dst = (d0, d1, d2)
        semi = (si0, si1, si2)
        semo = (so0, so1, so2)
        iot = lax.iota(jnp.int32, 16)
        rows = [iot + 16 * jj for jj in range(4)]

        def blk_of(kk):
            return w + _NW * kk

        def start_in(b, kk):
            pltpu.make_async_copy(
                tT.at[:, pl.ds(blk_of(kk) * 128, 128)], src[b], semi[b]
            ).start()

        def wait_in(b):
            pltpu.make_async_copy(
                tT.at[:, pl.ds(0, 128)], src[b], semi[b]).wait()

        def start_out(b, kk):
            pltpu.make_async_copy(
                dst[b], R.at[pl.ds(blk_of(kk) * (128 // 2), 128 // 2)],
                semo[b]).start()

        def wait_out(b):
            pltpu.make_async_copy(
                dst[b], R.at[pl.ds(0, 128 // 2)], semo[b]).wait()

        def transpose_block(sref, dref, nrows):
            @plsc.parallel_loop(0, D, 1, unroll=4)
            def _(d):
                dv = jnp.full((16,), 0, jnp.int32) + d
                base = lax.bitwise_and(iot + dv, 127)
                for m in range(8):
                    colv = lax.bitwise_and(base + 16 * m, 127)
                    g = plsc.load_gather(sref, [dv, colv])
                    rot[d, pl.ds(16 * m, 16)] = g

            @plsc.parallel_loop(0, nrows, 1, unroll=4)
            def _(p):
                for half in range(2):
                    vspl = jnp.full((16,), 0, jnp.int32) + (2 * p + half)
                    for jj in range(D // 16):
                        colv = lax.bitwise_and(vspl - rows[jj], 127)
                        g = plsc.load_gather(rot, [rows[jj], colv])
                        dref[p, pl.ds((half * (D // 16) + jj) * 16, 16)] = (
                            g * SCALE)

        start_in(0, 0)
        start_in(1, 1)

        def outer(o, carry):
            for b in (0, 1, 2):
                kk = 3 * o + b
                b2 = (b + 2) % 3

                @pl.when((kk + 2 <= KMAX - 1)
                         & (blk_of(kk + 2) < NB_FULL))
                def _():
                    start_in(b2, kk + 2)

                @pl.when((kk <= KMAX - 1) & (blk_of(kk) < NB_FULL))
                def _():
                    wait_in(b)

                    @pl.when(kk >= 3)
                    def _():
                        wait_out(b)

                    transpose_block(src[b], dst[b], 64)
                    start_out(b, kk)
            return carry

        lax.fori_loop(0, (KMAX + 2) // 3, outer, 0)
        wait_out(0)
        wait_out(1)
        wait_out(2)

        if TAIL:
            @pl.when(w == tail_w)
            def _():
                pltpu.sync_copy(tail128, s0)
                transpose_block(s0, d0, TAIL // 2)
                pltpu.sync_copy(
                    d0.at[pl.ds(0, TAIL // 2)],
                    R.at[pl.ds(NB_FULL * (128 // 2), TAIL // 2)])

    return k


@functools.lru_cache(maxsize=None)
def _make_gather(S_LEN: int, B: int, V: int, D: int):
    NBB = B // 128
    assert NBB == _NW and S_LEN % 2 == 0 and D == 64
    mesh = plsc.VectorSubcoreMesh(core_axis_name="c", subcore_axis_name="s")

    @functools.partial(
        pl.kernel,
        mesh=mesh,
        out_type=jax.ShapeDtypeStruct((S_LEN, D // 8, _NW, 8 * 128),
                                      jnp.float32),
        scratch_types=[
            pltpu.VMEM((S_LEN, 128), jnp.int32),
            pltpu.VMEM((128, D), jnp.float32),
            pltpu.VMEM((128, D), jnp.float32),
            pltpu.VMEM((128, D), jnp.float32),
            pltpu.VMEM((D // 8, 8 * 128), jnp.float32),
            pltpu.VMEM((D // 8, 8 * 128), jnp.float32),
            pltpu.VMEM((D // 8, 8 * 128), jnp.float32),
            pltpu.VMEM((128, D), jnp.float32),
            pltpu.SemaphoreType.DMA,
            pltpu.SemaphoreType.DMA,
            pltpu.SemaphoreType.DMA,
            pltpu.SemaphoreType.DMA,
            pltpu.SemaphoreType.DMA,
            pltpu.SemaphoreType.DMA,
            pltpu.SemaphoreType.DMA,
        ],
        compiler_params=pltpu.CompilerParams(use_tc_tiling_on_sc=False, needs_layout_passes=False),
    )
    def k(xT, R, O, xall, g0, g1, g2, sl0, sl1, sl2, grot, sx,
          sg0, sg1, sg2, so0, so1, so2):
        w = lax.axis_index("s") * _NC + lax.axis_index("c")
        G = (g0, g1, g2)
        S = (sl0, sl1, sl2)
        semg = (sg0, sg1, sg2)
        semo = (so0, so1, so2)
        iot = lax.iota(jnp.int32, 16)
        rowsc = [iot + 16 * c for c in range(8)]

        pltpu.make_async_copy(
            xT.at[:, pl.ds(128 * w, 128)], xall, sx).start()
        pltpu.make_async_copy(
            xT.at[:, pl.ds(0, 128)], xall, sx).wait()

        def start_gather(b, s):
            pltpu.make_async_copy(R.at[xall.at[s]], G[b], semg[b]).start()

        def wait_gather(b):
            pltpu.make_async_copy(R.at[xall.at[0]], G[b], semg[b]).wait()

        def start_out(b, s):
            pltpu.make_async_copy(
                S[b], O.at[s, :, w, :], semo[b]).start()

        def wait_out(b):
            pltpu.make_async_copy(
                S[b], O.at[0, :, w, :], semo[b]).wait()

        def transpose_unit(b):
            @plsc.parallel_loop(0, 128, 1, unroll=4)
            def _(r):
                rv = jnp.full((16,), 0, jnp.int32) + r
                base = lax.bitwise_and(iot + rv, D - 1)
                for m in range(D // 16):
                    colv = lax.bitwise_and(base + 16 * m, D - 1)
                    g = plsc.load_gather(G[b], [rv, colv])
                    grot[r, pl.ds(16 * m, 16)] = g

            @plsc.parallel_loop(0, D, 1, unroll=4)
            def _(d):
                dv = jnp.full((16,), 0, jnp.int32) + d
                dhi = d // 8
                dlo = d - dhi * 8
                for c in range(8):
                    colv = lax.bitwise_and(dv - rowsc[c], D - 1)
                    g = plsc.load_gather(grot, [rowsc[c], colv])
                    S[b][dhi, pl.ds(dlo * 128 + 16 * c, 16)] = g

        start_gather(0, 0)
        start_gather(1, 1)

        def outer(o, carry):
            for b in (0, 1, 2):
                s = 3 * o + b
                b2 = (b + 2) % 3

                @pl.when(s + 2 <= S_LEN - 1)
                def _():
                    start_gather(b2, s + 2)

                @pl.when(s <= S_LEN - 1)
                def _():
                    wait_gather(b)

                    @pl.when(s >= 3)
                    def _():
                        wait_out(b)

                    transpose_unit(b)
                    start_out(b, s)
            return carry

        lax.fori_loop(0, (S_LEN + 2) // 3, outer, 0)
        wait_out(0)
        wait_out(1)
        wait_out(2)

    return k


def kernel(x, table):
    B, S_LEN = x.shape
    V, D = table.shape
    tT = table.T
    tail = V % 128
    tail128 = jnp.pad(table[V - tail:], ((0, 128 - tail), (0, 0))).T
    R = _make_transpose(V, D)(tT, tail128)
    xT = x.astype(jnp.int32).T
    O = _make_gather(S_LEN, B, V, D)(xT, R.reshape(V, D))
    o5 = O.reshape(S_LEN, D // 8, _NW, 8, 128)
    return o5.transpose(2, 4, 0, 1, 3).reshape(B, S_LEN, D)
